# baseline (speedup 1.0000x reference)
import functools

import jax
import jax.numpy as jnp
from jax import lax
from jax.experimental import pallas as pl
from jax.experimental.pallas import tpu as pltpu
from jax.experimental.pallas import tpu_sc as plsc

B = 1024
C = 20
D = 16
V = 100000

NC = 2
NS = 16
NW = NC * NS

NGROUPS = B // 16
VSPLIT = 49920
VLEN0 = VSPLIT
VLEN1 = V - VSPLIT

_mesh = plsc.VectorSubcoreMesh(core_axis_name="c", subcore_axis_name="s")


@functools.partial(
    pl.kernel,
    mesh=_mesh,
    out_type=jax.ShapeDtypeStruct((NC, D, B), jnp.float32),
    scratch_types=[
        pltpu.VMEM((VLEN1,), jnp.float32),
        pltpu.VMEM((C, B), jnp.int32),
        pltpu.VMEM((B,), jnp.float32),
        pltpu.SemaphoreType.DMA,
    ],
    compiler_params=pltpu.CompilerParams(
        use_tc_tiling_on_sc=True, needs_layout_passes=False
    ),
)
def _pool(idx_hbm, tablet_hbm, out_hbm, row_v, idx_v, acc_v, sem):
    d = lax.axis_index("s")
    j = lax.axis_index("c")

    @pl.when(j == 0)
    def _():
        pltpu.sync_copy(tablet_hbm.at[d, pl.ds(0, VLEN0)],
                        row_v.at[pl.ds(0, VLEN0)])

    @pl.when(j == 1)
    def _():
        pltpu.sync_copy(tablet_hbm.at[d, pl.ds(VSPLIT, VLEN1)], row_v)

    pltpu.sync_copy(idx_hbm, idx_v)

    lower = j * VSPLIT
    upper = lower + VLEN0 + j * (VLEN1 - VLEN0)

    def body(g, _):
        base = g * 16
        acc = jnp.zeros((16,), jnp.float32)
        for c in range(C):
            idx16 = idx_v[c, pl.ds(base, 16)]
            mask = (idx16 >= lower) & (idx16 < upper)
            local = jnp.where(mask, idx16 - lower, 0)
            vals = plsc.load_gather(row_v, [local], mask=mask)
            acc = acc + jnp.where(mask, vals, 0.0)
        acc_v[pl.ds(base, 16)] = acc * (1.0 / C)
        return _

    lax.fori_loop(0, NGROUPS, body, None)

    pltpu.sync_copy(acc_v, out_hbm.at[j, d])


VB = 2048
_GRID = pl.cdiv(V, VB)


def _proj_body(xt_ref, wt_ref, b_ref, o_ref):
    acc = lax.dot_general(
        wt_ref[...],
        xt_ref[...],
        dimension_numbers=(((0,), (0,)), ((), ())),
        preferred_element_type=jnp.float32,
    )
    o_ref[...] = acc + b_ref[...].T


_proj = pl.pallas_call(
    _proj_body,
    grid=(_GRID,),
    in_specs=[
        pl.BlockSpec((D, B), lambda i: (0, 0)),
        pl.BlockSpec((D, VB), lambda i: (0, i)),
        pl.BlockSpec((1, VB), lambda i: (0, i)),
    ],
    out_specs=pl.BlockSpec((VB, B), lambda i: (i, 0)),
    out_shape=jax.ShapeDtypeStruct((V, B), jnp.float32),
)


def kernel(context_words, emb_table, linear_w, linear_b):
    idx_cm = context_words.astype(jnp.int32).T
    partial = _pool(idx_cm, emb_table.T)
    mean_t = partial[0] + partial[1]
    out_t = _proj(mean_t, linear_w.T, linear_b.reshape(1, V))
    return out_t.T


# speedup vs baseline: 1.0129x; 1.0129x over previous
import functools

import jax
import jax.numpy as jnp
from jax import lax
from jax.experimental import pallas as pl
from jax.experimental.pallas import tpu as pltpu
from jax.experimental.pallas import tpu_sc as plsc

B = 1024
C = 20
D = 16
V = 100000

NC = 2
NS = 16
NW = NC * NS

B_PER_H = B // NC
NGROUPS = B_PER_H // 16

_mesh = plsc.VectorSubcoreMesh(core_axis_name="c", subcore_axis_name="s")


@functools.partial(
    pl.kernel,
    mesh=_mesh,
    out_type=jax.ShapeDtypeStruct((D, B), jnp.float32),
    scratch_types=[
        pltpu.VMEM((V,), jnp.float32),
        pltpu.VMEM((C, B_PER_H), jnp.int32),
        pltpu.VMEM((B_PER_H,), jnp.float32),
        pltpu.SemaphoreType.DMA,
    ],
    compiler_params=pltpu.CompilerParams(
        use_tc_tiling_on_sc=True, needs_layout_passes=False
    ),
)
def _pool(idx_hbm, tablet_hbm, out_hbm, row_v, idx_v, acc_v, sem):
    d = lax.axis_index("s")
    h = lax.axis_index("c")

    row_cp = pltpu.async_copy(tablet_hbm.at[d], row_v, sem)
    pltpu.sync_copy(idx_hbm.at[:, pl.ds(h * B_PER_H, B_PER_H)], idx_v)
    row_cp.wait()

    def body(g, _):
        base = g * 16
        acc = jnp.zeros((16,), jnp.float32)
        for c in range(C):
            idx16 = idx_v[c, pl.ds(base, 16)]
            acc = acc + plsc.load_gather(row_v, [idx16])
        acc_v[pl.ds(base, 16)] = acc * (1.0 / C)
        return _

    lax.fori_loop(0, NGROUPS, body, None)

    pltpu.sync_copy(acc_v, out_hbm.at[d, pl.ds(h * B_PER_H, B_PER_H)])


VB = 2048
_GRID = pl.cdiv(V, VB)


def _proj_body(xt_ref, wt_ref, b_ref, o_ref):
    acc = lax.dot_general(
        wt_ref[...],
        xt_ref[...],
        dimension_numbers=(((0,), (0,)), ((), ())),
        preferred_element_type=jnp.float32,
    )
    o_ref[...] = acc + b_ref[...].T


_proj = pl.pallas_call(
    _proj_body,
    grid=(_GRID,),
    in_specs=[
        pl.BlockSpec((D, B), lambda i: (0, 0)),
        pl.BlockSpec((D, VB), lambda i: (0, i)),
        pl.BlockSpec((1, VB), lambda i: (0, i)),
    ],
    out_specs=pl.BlockSpec((VB, B), lambda i: (i, 0)),
    out_shape=jax.ShapeDtypeStruct((V, B), jnp.float32),
)


def kernel(context_words, emb_table, linear_w, linear_b):
    idx_cm = context_words.astype(jnp.int32).T
    mean_t = _pool(idx_cm, emb_table.T)
    out_t = _proj(mean_t, linear_w.T, linear_b.reshape(1, V))
    return out_t.T
